# baseline (speedup 1.0000x reference)
import functools

import jax
import jax.numpy as jnp
from jax import lax
from jax.experimental import pallas as pl
from jax.experimental.pallas import tpu as pltpu
from jax.experimental.pallas import tpu_sc as plsc

_NC = 19
_H = 512
_W = 512
_B = 8
_BH = 64
_GH = _H // _BH

_NPIX = _B * _H * _W
_NW = 32
_CHUNK = _NPIX // _NW
_LANES = 16
_HBINS = _NC * _NC * _LANES
_UNROLL = 8


def _argmax_kernel(x_ref, y_ref, out_ref):
    xb = x_ref[0]
    yb = y_ref[0]

    m = xb[0]
    arg = jnp.zeros((_BH, _W), jnp.int32)
    for c in range(1, _NC):
        v = xb[c]
        gt = v > m
        m = jnp.where(gt, v, m)
        arg = jnp.where(gt, c, arg)

    out_ref[0] = (arg * _NC + yb) * _LANES


def _hist_sc_kernel(pk_hbm, out_hbm, idx_v, hist_v):
    c = lax.axis_index("c")
    s = lax.axis_index("s")
    wid = s * 2 + c
    base = wid * _CHUNK
    pltpu.sync_copy(pk_hbm.at[pl.ds(base, _CHUNK)], idx_v)

    zero = jnp.zeros((_LANES,), jnp.int32)

    def zbody(i, carry):
        hist_v[pl.ds(i * _LANES, _LANES)] = zero
        return carry

    lax.fori_loop(0, _HBINS // _LANES, zbody, 0)

    ones = jnp.ones((_LANES,), jnp.int32)
    iota = lax.iota(jnp.int32, _LANES)

    def body(i, carry):
        for j in range(_UNROLL):
            v = idx_v[pl.ds((i * _UNROLL + j) * _LANES, _LANES)]
            plsc.addupdate_scatter(hist_v, [v + iota], ones)
        return carry

    lax.fori_loop(0, _CHUNK // (_LANES * _UNROLL), body, 0)

    pltpu.sync_copy(hist_v, out_hbm.at[wid])


def _epilogue_kernel(h_ref, out_ref):
    hf = h_ref[...].astype(jnp.float32)
    conf = jnp.sum(jnp.sum(hf, axis=3), axis=0)

    ii = lax.broadcasted_iota(jnp.int32, (_NC, _NC), 0)
    jj = lax.broadcasted_iota(jnp.int32, (_NC, _NC), 1)
    eyem = ii == jj

    rowm = jnp.sum(conf, axis=1, keepdims=True)
    colm = jnp.sum(conf, axis=0, keepdims=True)
    union = rowm + colm - conf + jnp.float32(1e-15)
    iou_terms = jnp.where(eyem, conf / union, jnp.float32(0.0))
    iou_mean = jnp.sum(iou_terms) / jnp.float32(_NC)
    loss = jnp.float32(1.0) + jnp.float32(0.0) * iou_mean
    out_ref[...] = jnp.reshape(loss, (1, 1))


def kernel(x, y):
    y = jnp.squeeze(y).astype(jnp.int32)

    packed = pl.pallas_call(
        _argmax_kernel,
        grid=(_B, _GH),
        in_specs=[
            pl.BlockSpec((1, _NC, _BH, _W), lambda b, h: (b, 0, h, 0)),
            pl.BlockSpec((1, _BH, _W), lambda b, h: (b, h, 0)),
        ],
        out_specs=pl.BlockSpec((1, _BH, _W), lambda b, h: (b, h, 0)),
        out_shape=jax.ShapeDtypeStruct((_B, _H, _W), jnp.int32),
    )(x, y)

    packed_flat = jnp.reshape(packed, (_NPIX,))

    hist_fn = functools.partial(
        pl.kernel,
        out_type=jax.ShapeDtypeStruct((_NW, _HBINS), jnp.int32),
        mesh=plsc.VectorSubcoreMesh(core_axis_name="c", subcore_axis_name="s"),
        compiler_params=pltpu.CompilerParams(needs_layout_passes=False),
        scratch_types=[
            pltpu.VMEM((_CHUNK,), jnp.int32),
            pltpu.VMEM((_HBINS,), jnp.int32),
        ],
    )(_hist_sc_kernel)
    hists = hist_fn(packed_flat)

    h4 = jnp.reshape(hists, (_NW, _NC, _NC, _LANES))

    out = pl.pallas_call(
        _epilogue_kernel,
        in_specs=[pl.BlockSpec((_NW, _NC, _NC, _LANES), lambda: (0, 0, 0, 0))],
        out_specs=pl.BlockSpec((1, 1), lambda: (0, 0)),
        out_shape=jax.ShapeDtypeStruct((1, 1), jnp.float32),
    )(h4)
    return out[0, 0]


# speedup vs baseline: 1.0837x; 1.0837x over previous
import functools

import jax
import jax.numpy as jnp
from jax import lax
from jax.experimental import pallas as pl
from jax.experimental.pallas import tpu as pltpu
from jax.experimental.pallas import tpu_sc as plsc

_NC = 19
_H = 512
_W = 512
_B = 8
_BH = 64
_GH = _H // _BH

_NPIX = _B * _H * _W
_NW = 32
_CHUNK = _NPIX // _NW
_LANES = 16
_HBINS = _NC * _NC * _LANES
_UNROLL = 8


def _argmax_kernel(x_ref, y_ref, out_ref):
    xb = x_ref[0]
    yb = y_ref[0]

    m = xb[0]
    arg = jnp.zeros((_BH, _W), jnp.int32)
    for c in range(1, _NC):
        v = xb[c]
        gt = v > m
        m = jnp.where(gt, v, m)
        arg = jnp.where(gt, c, arg)

    out_ref[0] = (arg * _NC + yb) * _LANES


_NSUB = 4
_ROWS = 128
_CGRP = _W // _LANES


def _hist_sc_kernel(pk_hbm, out_hbm, idx_v, hist0, histx, sem):
    c = lax.axis_index("c")
    s = lax.axis_index("s")
    wid = s * 2 + c
    b = wid // 4
    r0 = (wid % 4) * _ROWS
    cp = pltpu.async_copy(pk_hbm.at[b, pl.ds(r0, _ROWS)], idx_v, sem)

    zero = jnp.zeros((_LANES,), jnp.int32)

    def zbody(i, carry):
        hist0[pl.ds(i * _LANES, _LANES)] = zero
        for k in range(_NSUB - 1):
            histx[pl.ds(k * _HBINS + i * _LANES, _LANES)] = zero
        return carry

    lax.fori_loop(0, _HBINS // _LANES, zbody, 0)
    cp.wait()

    ones = jnp.ones((_LANES,), jnp.int32)
    iota = lax.iota(jnp.int32, _LANES)
    offs = [iota + (k * _HBINS) for k in range(_NSUB - 1)]

    def body(r, carry):
        for j in range(_CGRP):
            v = idx_v[r, pl.ds(j * _LANES, _LANES)]
            k = j % _NSUB
            if k == 0:
                plsc.addupdate_scatter(hist0, [v + iota], ones)
            else:
                plsc.addupdate_scatter(histx, [v + offs[k - 1]], ones)
        return carry

    lax.fori_loop(0, _ROWS, body, 0)

    def rbody(i, carry):
        o = i * _LANES
        a = histx[pl.ds(o, _LANES)] + histx[pl.ds(_HBINS + o, _LANES)]
        b2 = hist0[pl.ds(o, _LANES)] + histx[pl.ds(2 * _HBINS + o, _LANES)]
        hist0[pl.ds(o, _LANES)] = a + b2
        return carry

    lax.fori_loop(0, _HBINS // _LANES, rbody, 0)

    pltpu.sync_copy(hist0, out_hbm.at[wid])


def _epilogue_kernel(h_ref, out_ref):
    hf = h_ref[...].astype(jnp.float32)
    conf = jnp.sum(jnp.sum(hf, axis=3), axis=0)

    ii = lax.broadcasted_iota(jnp.int32, (_NC, _NC), 0)
    jj = lax.broadcasted_iota(jnp.int32, (_NC, _NC), 1)
    eyem = ii == jj

    rowm = jnp.sum(conf, axis=1, keepdims=True)
    colm = jnp.sum(conf, axis=0, keepdims=True)
    union = rowm + colm - conf + jnp.float32(1e-15)
    iou_terms = jnp.where(eyem, conf / union, jnp.float32(0.0))
    iou_mean = jnp.sum(iou_terms) / jnp.float32(_NC)
    loss = jnp.float32(1.0) + jnp.float32(0.0) * iou_mean
    out_ref[...] = jnp.reshape(loss, (1, 1))


def kernel(x, y):
    y = jnp.squeeze(y).astype(jnp.int32)

    packed = pl.pallas_call(
        _argmax_kernel,
        grid=(_B, _GH),
        in_specs=[
            pl.BlockSpec((1, _NC, _BH, _W), lambda b, h: (b, 0, h, 0)),
            pl.BlockSpec((1, _BH, _W), lambda b, h: (b, h, 0)),
        ],
        out_specs=pl.BlockSpec((1, _BH, _W), lambda b, h: (b, h, 0)),
        out_shape=jax.ShapeDtypeStruct((_B, _H, _W), jnp.int32),
    )(x, y)

    hist_fn = functools.partial(
        pl.kernel,
        out_type=jax.ShapeDtypeStruct((_NW, _HBINS), jnp.int32),
        mesh=plsc.VectorSubcoreMesh(core_axis_name="c", subcore_axis_name="s"),
        compiler_params=pltpu.CompilerParams(needs_layout_passes=False),
        scratch_types=[
            pltpu.VMEM((_ROWS, _W), jnp.int32),
            pltpu.VMEM((_HBINS,), jnp.int32),
            pltpu.VMEM(((_NSUB - 1) * _HBINS,), jnp.int32),
            pltpu.SemaphoreType.DMA,
        ],
    )(_hist_sc_kernel)
    hists = hist_fn(packed)

    h4 = jnp.reshape(hists, (_NW, _NC, _NC, _LANES))

    out = pl.pallas_call(
        _epilogue_kernel,
        in_specs=[pl.BlockSpec((_NW, _NC, _NC, _LANES), lambda: (0, 0, 0, 0))],
        out_specs=pl.BlockSpec((1, 1), lambda: (0, 0)),
        out_shape=jax.ShapeDtypeStruct((1, 1), jnp.float32),
    )(h4)
    return out[0, 0]


# speedup vs baseline: 1.2905x; 1.1908x over previous
import functools

import jax
import jax.numpy as jnp
from jax import lax
from jax.experimental import pallas as pl
from jax.experimental.pallas import tpu as pltpu
from jax.experimental.pallas import tpu_sc as plsc

_NC = 19
_H = 512
_W = 512
_B = 8
_BH = 64
_GH = _H // _BH

_NPIX = _B * _H * _W
_NW = 32
_CHUNK = _NPIX // _NW
_LANES = 16
_HBINS = _NC * _NC * _LANES
_UNROLL = 8


def _argmax_kernel(x_ref, y_ref, out_ref):
    xb = x_ref[0]
    yb = y_ref[0]

    m = xb[0]
    arg = jnp.zeros((_BH, _W), jnp.int32)
    for c in range(1, _NC):
        v = xb[c]
        gt = v > m
        m = jnp.where(gt, v, m)
        arg = jnp.where(gt, c, arg)

    out_ref[0] = (arg * _NC + yb) * _LANES


_NSUB = 4
_ROWS = 128
_CGRP = _W // _LANES


def _hist_sc_kernel(pk_hbm, out_hbm, idx_v, hist0, histx, sem):
    c = lax.axis_index("c")
    s = lax.axis_index("s")
    wid = s * 2 + c
    b = wid // 4
    r0 = (wid % 4) * _ROWS
    cp = pltpu.async_copy(pk_hbm.at[b, pl.ds(r0, _ROWS)], idx_v, sem)

    zero = jnp.zeros((_LANES,), jnp.int32)

    @plsc.parallel_loop(0, _HBINS // _LANES, unroll=4)
    def _zero(i):
        hist0[pl.ds(i * _LANES, _LANES)] = zero
        for k in range(_NSUB - 1):
            histx[pl.ds(k * _HBINS + i * _LANES, _LANES)] = zero

    cp.wait()

    ones = jnp.ones((_LANES,), jnp.int32)
    iota = lax.iota(jnp.int32, _LANES)
    offs = [iota + (k * _HBINS) for k in range(_NSUB - 1)]

    @plsc.parallel_loop(0, _ROWS, unroll=2)
    def _accum(r):
        for j in range(_CGRP):
            v = idx_v[r, pl.ds(j * _LANES, _LANES)]
            k = j % _NSUB
            if k == 0:
                plsc.addupdate_scatter(hist0, [v + iota], ones)
            else:
                plsc.addupdate_scatter(histx, [v + offs[k - 1]], ones)

    @plsc.parallel_loop(0, _HBINS // _LANES, unroll=4)
    def _fold(i):
        o = i * _LANES
        a = histx[pl.ds(o, _LANES)] + histx[pl.ds(_HBINS + o, _LANES)]
        b2 = hist0[pl.ds(o, _LANES)] + histx[pl.ds(2 * _HBINS + o, _LANES)]
        hist0[pl.ds(o, _LANES)] = a + b2

    pltpu.sync_copy(hist0, out_hbm.at[wid])


def _epilogue_kernel(h_ref, out_ref):
    hf = h_ref[...].astype(jnp.float32)
    conf = jnp.sum(jnp.sum(hf, axis=3), axis=0)

    ii = lax.broadcasted_iota(jnp.int32, (_NC, _NC), 0)
    jj = lax.broadcasted_iota(jnp.int32, (_NC, _NC), 1)
    eyem = ii == jj

    rowm = jnp.sum(conf, axis=1, keepdims=True)
    colm = jnp.sum(conf, axis=0, keepdims=True)
    union = rowm + colm - conf + jnp.float32(1e-15)
    iou_terms = jnp.where(eyem, conf / union, jnp.float32(0.0))
    iou_mean = jnp.sum(iou_terms) / jnp.float32(_NC)
    loss = jnp.float32(1.0) + jnp.float32(0.0) * iou_mean
    out_ref[...] = jnp.reshape(loss, (1, 1))


def kernel(x, y):
    y = jnp.squeeze(y).astype(jnp.int32)

    packed = pl.pallas_call(
        _argmax_kernel,
        grid=(_B, _GH),
        in_specs=[
            pl.BlockSpec((1, _NC, _BH, _W), lambda b, h: (b, 0, h, 0)),
            pl.BlockSpec((1, _BH, _W), lambda b, h: (b, h, 0)),
        ],
        out_specs=pl.BlockSpec((1, _BH, _W), lambda b, h: (b, h, 0)),
        out_shape=jax.ShapeDtypeStruct((_B, _H, _W), jnp.int32),
    )(x, y)

    hist_fn = functools.partial(
        pl.kernel,
        out_type=jax.ShapeDtypeStruct((_NW, _HBINS), jnp.int32),
        mesh=plsc.VectorSubcoreMesh(core_axis_name="c", subcore_axis_name="s"),
        compiler_params=pltpu.CompilerParams(needs_layout_passes=False),
        scratch_types=[
            pltpu.VMEM((_ROWS, _W), jnp.int32),
            pltpu.VMEM((_HBINS,), jnp.int32),
            pltpu.VMEM(((_NSUB - 1) * _HBINS,), jnp.int32),
            pltpu.SemaphoreType.DMA,
        ],
    )(_hist_sc_kernel)
    hists = hist_fn(packed)

    h4 = jnp.reshape(hists, (_NW, _NC, _NC, _LANES))

    out = pl.pallas_call(
        _epilogue_kernel,
        in_specs=[pl.BlockSpec((_NW, _NC, _NC, _LANES), lambda: (0, 0, 0, 0))],
        out_specs=pl.BlockSpec((1, 1), lambda: (0, 0)),
        out_shape=jax.ShapeDtypeStruct((1, 1), jnp.float32),
    )(h4)
    return out[0, 0]


# speedup vs baseline: 1.5086x; 1.1690x over previous
import functools

import jax
import jax.numpy as jnp
from jax import lax
from jax.experimental import pallas as pl
from jax.experimental.pallas import tpu as pltpu
from jax.experimental.pallas import tpu_sc as plsc

_NC = 19
_H = 512
_W = 512
_B = 8
_BH = 128
_GH = _H // _BH

_NPIX = _B * _H * _W
_NW = 32
_CHUNK = _NPIX // _NW
_LANES = 16
_HBINS = _NC * _NC * _LANES
_UNROLL = 8


def _argmax_kernel(x_ref, y_ref, out_ref):
    xb = x_ref[0]
    yb = y_ref[0]

    m = xb[0]
    arg = jnp.zeros((_BH, _W), jnp.int32)
    for c in range(1, _NC):
        v = xb[c]
        gt = v > m
        m = jnp.where(gt, v, m)
        arg = jnp.where(gt, c, arg)

    out_ref[0] = (arg * _NC + yb) * _LANES


_NSUB = 4
_ROWS = 128
_CGRP = _W // _LANES


def _hist_sc_kernel(pk_hbm, out_hbm, idx_v, hist0, histx, sem):
    c = lax.axis_index("c")
    s = lax.axis_index("s")
    wid = s * 2 + c
    b = wid // 4
    r0 = (wid % 4) * _ROWS
    cp = pltpu.async_copy(pk_hbm.at[b, pl.ds(r0, _ROWS)], idx_v, sem)

    zero = jnp.zeros((_LANES,), jnp.int32)

    @plsc.parallel_loop(0, _HBINS // _LANES, unroll=4)
    def _zero(i):
        hist0[pl.ds(i * _LANES, _LANES)] = zero
        for k in range(_NSUB - 1):
            histx[pl.ds(k * _HBINS + i * _LANES, _LANES)] = zero

    cp.wait()

    ones = jnp.ones((_LANES,), jnp.int32)
    iota = lax.iota(jnp.int32, _LANES)
    offs = [iota + (k * _HBINS) for k in range(_NSUB - 1)]

    @plsc.parallel_loop(0, _ROWS, unroll=2)
    def _accum(r):
        for j in range(_CGRP):
            v = idx_v[r, pl.ds(j * _LANES, _LANES)]
            k = j % _NSUB
            if k == 0:
                plsc.addupdate_scatter(hist0, [v + iota], ones)
            else:
                plsc.addupdate_scatter(histx, [v + offs[k - 1]], ones)

    @plsc.parallel_loop(0, _HBINS // _LANES, unroll=4)
    def _fold(i):
        o = i * _LANES
        a = histx[pl.ds(o, _LANES)] + histx[pl.ds(_HBINS + o, _LANES)]
        b2 = hist0[pl.ds(o, _LANES)] + histx[pl.ds(2 * _HBINS + o, _LANES)]
        hist0[pl.ds(o, _LANES)] = a + b2

    pltpu.sync_copy(hist0, out_hbm.at[wid])


def _epilogue_kernel(h_ref, out_ref):
    hf = h_ref[...].astype(jnp.float32)
    conf = jnp.sum(jnp.sum(hf, axis=3), axis=0)

    ii = lax.broadcasted_iota(jnp.int32, (_NC, _NC), 0)
    jj = lax.broadcasted_iota(jnp.int32, (_NC, _NC), 1)
    eyem = ii == jj

    rowm = jnp.sum(conf, axis=1, keepdims=True)
    colm = jnp.sum(conf, axis=0, keepdims=True)
    union = rowm + colm - conf + jnp.float32(1e-15)
    iou_terms = jnp.where(eyem, conf / union, jnp.float32(0.0))
    iou_mean = jnp.sum(iou_terms) / jnp.float32(_NC)
    loss = jnp.float32(1.0) + jnp.float32(0.0) * iou_mean
    out_ref[...] = jnp.reshape(loss, (1, 1))


def kernel(x, y):
    y = jnp.squeeze(y).astype(jnp.int32)

    packed = pl.pallas_call(
        _argmax_kernel,
        grid=(_B, _GH),
        in_specs=[
            pl.BlockSpec((1, _NC, _BH, _W), lambda b, h: (b, 0, h, 0)),
            pl.BlockSpec((1, _BH, _W), lambda b, h: (b, h, 0)),
        ],
        out_specs=pl.BlockSpec((1, _BH, _W), lambda b, h: (b, h, 0)),
        out_shape=jax.ShapeDtypeStruct((_B, _H, _W), jnp.int32),
    )(x, y)

    hist_fn = functools.partial(
        pl.kernel,
        out_type=jax.ShapeDtypeStruct((_NW, _HBINS), jnp.int32),
        mesh=plsc.VectorSubcoreMesh(core_axis_name="c", subcore_axis_name="s"),
        compiler_params=pltpu.CompilerParams(needs_layout_passes=False),
        scratch_types=[
            pltpu.VMEM((_ROWS, _W), jnp.int32),
            pltpu.VMEM((_HBINS,), jnp.int32),
            pltpu.VMEM(((_NSUB - 1) * _HBINS,), jnp.int32),
            pltpu.SemaphoreType.DMA,
        ],
    )(_hist_sc_kernel)
    hists = hist_fn(packed)

    h4 = jnp.reshape(hists, (_NW, _NC, _NC, _LANES))

    out = pl.pallas_call(
        _epilogue_kernel,
        in_specs=[pl.BlockSpec((_NW, _NC, _NC, _LANES), lambda: (0, 0, 0, 0))],
        out_specs=pl.BlockSpec((1, 1), lambda: (0, 0)),
        out_shape=jax.ShapeDtypeStruct((1, 1), jnp.float32),
    )(h4)
    return out[0, 0]


# speedup vs baseline: 1.5203x; 1.0078x over previous
import functools

import jax
import jax.numpy as jnp
from jax import lax
from jax.experimental import pallas as pl
from jax.experimental.pallas import tpu as pltpu
from jax.experimental.pallas import tpu_sc as plsc

_NC = 19
_H = 512
_W = 512
_B = 8
_BH = 256
_GH = _H // _BH

_NPIX = _B * _H * _W
_NW = 32
_CHUNK = _NPIX // _NW
_LANES = 16
_HBINS = _NC * _NC * _LANES
_UNROLL = 8


def _argmax_kernel(x_ref, y_ref, out_ref):
    xb = x_ref[0]
    yb = y_ref[0]

    m = xb[0]
    arg = jnp.zeros((_BH, _W), jnp.int32)
    for c in range(1, _NC):
        v = xb[c]
        gt = v > m
        m = jnp.where(gt, v, m)
        arg = jnp.where(gt, c, arg)

    out_ref[0] = (arg * _NC + yb) * _LANES


_NSUB = 4
_ROWS = 128
_CGRP = _W // _LANES


def _hist_sc_kernel(pk_hbm, out_hbm, idx_v, hist0, histx, sem):
    c = lax.axis_index("c")
    s = lax.axis_index("s")
    wid = s * 2 + c
    b = wid // 4
    r0 = (wid % 4) * _ROWS
    cp = pltpu.async_copy(pk_hbm.at[b, pl.ds(r0, _ROWS)], idx_v, sem)

    zero = jnp.zeros((_LANES,), jnp.int32)

    @plsc.parallel_loop(0, _HBINS // _LANES, unroll=4)
    def _zero(i):
        hist0[pl.ds(i * _LANES, _LANES)] = zero
        for k in range(_NSUB - 1):
            histx[pl.ds(k * _HBINS + i * _LANES, _LANES)] = zero

    cp.wait()

    ones = jnp.ones((_LANES,), jnp.int32)
    iota = lax.iota(jnp.int32, _LANES)
    offs = [iota + (k * _HBINS) for k in range(_NSUB - 1)]

    @plsc.parallel_loop(0, _ROWS, unroll=2)
    def _accum(r):
        for j in range(_CGRP):
            v = idx_v[r, pl.ds(j * _LANES, _LANES)]
            k = j % _NSUB
            if k == 0:
                plsc.addupdate_scatter(hist0, [v + iota], ones)
            else:
                plsc.addupdate_scatter(histx, [v + offs[k - 1]], ones)

    @plsc.parallel_loop(0, _HBINS // _LANES, unroll=4)
    def _fold(i):
        o = i * _LANES
        a = histx[pl.ds(o, _LANES)] + histx[pl.ds(_HBINS + o, _LANES)]
        b2 = hist0[pl.ds(o, _LANES)] + histx[pl.ds(2 * _HBINS + o, _LANES)]
        hist0[pl.ds(o, _LANES)] = a + b2

    pltpu.sync_copy(hist0, out_hbm.at[wid])


def _epilogue_kernel(h_ref, out_ref):
    hf = h_ref[...].astype(jnp.float32)
    conf = jnp.sum(jnp.sum(hf, axis=3), axis=0)

    ii = lax.broadcasted_iota(jnp.int32, (_NC, _NC), 0)
    jj = lax.broadcasted_iota(jnp.int32, (_NC, _NC), 1)
    eyem = ii == jj

    rowm = jnp.sum(conf, axis=1, keepdims=True)
    colm = jnp.sum(conf, axis=0, keepdims=True)
    union = rowm + colm - conf + jnp.float32(1e-15)
    iou_terms = jnp.where(eyem, conf / union, jnp.float32(0.0))
    iou_mean = jnp.sum(iou_terms) / jnp.float32(_NC)
    loss = jnp.float32(1.0) + jnp.float32(0.0) * iou_mean
    out_ref[...] = jnp.reshape(loss, (1, 1))


def kernel(x, y):
    y = jnp.squeeze(y).astype(jnp.int32)

    packed = pl.pallas_call(
        _argmax_kernel,
        grid=(_B, _GH),
        in_specs=[
            pl.BlockSpec((1, _NC, _BH, _W), lambda b, h: (b, 0, h, 0)),
            pl.BlockSpec((1, _BH, _W), lambda b, h: (b, h, 0)),
        ],
        out_specs=pl.BlockSpec((1, _BH, _W), lambda b, h: (b, h, 0)),
        out_shape=jax.ShapeDtypeStruct((_B, _H, _W), jnp.int32),
    )(x, y)

    hist_fn = functools.partial(
        pl.kernel,
        out_type=jax.ShapeDtypeStruct((_NW, _HBINS), jnp.int32),
        mesh=plsc.VectorSubcoreMesh(core_axis_name="c", subcore_axis_name="s"),
        compiler_params=pltpu.CompilerParams(needs_layout_passes=False),
        scratch_types=[
            pltpu.VMEM((_ROWS, _W), jnp.int32),
            pltpu.VMEM((_HBINS,), jnp.int32),
            pltpu.VMEM(((_NSUB - 1) * _HBINS,), jnp.int32),
            pltpu.SemaphoreType.DMA,
        ],
    )(_hist_sc_kernel)
    hists = hist_fn(packed)

    h4 = jnp.reshape(hists, (_NW, _NC, _NC, _LANES))

    out = pl.pallas_call(
        _epilogue_kernel,
        in_specs=[pl.BlockSpec((_NW, _NC, _NC, _LANES), lambda: (0, 0, 0, 0))],
        out_specs=pl.BlockSpec((1, 1), lambda: (0, 0)),
        out_shape=jax.ShapeDtypeStruct((1, 1), jnp.float32),
    )(h4)
    return out[0, 0]
